# Initial kernel scaffold; baseline (speedup 1.0000x reference)
import jax
import jax.numpy as jnp
from jax.experimental import pallas as pl


def kernel(distance_matrix, direction_matrix, track_connectivity, dist_table, dir_table, track_bias):
    raise NotImplementedError("write your pallas kernel here")


# speedup vs baseline: 21.8114x; 21.8114x over previous
import functools

import jax
import jax.numpy as jnp
from jax import lax
from jax.experimental import pallas as pl
from jax.experimental.pallas import tpu as pltpu
from jax.experimental.pallas import tpu_sc as plsc

_H = 16
_N = 1024
_NP = _N * _N
_NW = 32
_PW = _NP // _NW
_CHP = 4096
_NCH = _PW // _CHP
_NSL = _CHP // 16
_TROW = 96
_TSZ = _H * _TROW + _H


@functools.partial(
    pl.kernel,
    mesh=plsc.VectorSubcoreMesh(core_axis_name="c", subcore_axis_name="s"),
    out_type=jax.ShapeDtypeStruct((_H, _NP), jnp.float32),
    scratch_types=[
        pltpu.VMEM((_TSZ,), jnp.float32),
        pltpu.VMEM((_CHP,), jnp.int32),
        pltpu.VMEM((_CHP,), jnp.int32),
        pltpu.VMEM((_CHP,), jnp.float32),
        pltpu.VMEM((_H, _CHP), jnp.float32),
    ],
    compiler_params=pltpu.CompilerParams(needs_layout_passes=False),
)
def _bias_kernel(ctab_hbm, dist_hbm, dir_hbm, track_hbm, out_hbm,
                 ctab_v, dist_v, dir_v, track_v, out_v):
    wid = lax.axis_index("s") * 2 + lax.axis_index("c")
    pltpu.sync_copy(ctab_hbm, ctab_v)
    tb_splat = [
        plsc.load_gather(ctab_v, [jnp.full((16,), _H * _TROW + h, jnp.int32)])
        for h in range(_H)
    ]
    base0 = wid * _PW

    def chunk_body(ci, carry):
        base = base0 + ci * _CHP
        pltpu.sync_copy(dist_hbm.at[pl.ds(base, _CHP)], dist_v)
        pltpu.sync_copy(dir_hbm.at[pl.ds(base, _CHP)], dir_v)
        pltpu.sync_copy(track_hbm.at[pl.ds(base, _CHP)], track_v)

        def slice_body(s, c2):
            off = s * 16
            d = dist_v[pl.ds(off, 16)]
            r = dir_v[pl.ds(off, 16)]
            t = track_v[pl.ds(off, 16)]
            c = jnp.minimum(d, 12) * 7 + r
            for h in range(_H):
                g = plsc.load_gather(ctab_v, [c + h * _TROW])
                out_v[h, pl.ds(off, 16)] = g + t * tb_splat[h]
            return c2

        lax.fori_loop(0, _NSL, slice_body, 0)
        for h in range(_H):
            pltpu.sync_copy(out_v.at[h], out_hbm.at[h, pl.ds(base, _CHP)])
        return carry

    lax.fori_loop(0, _NCH, chunk_body, 0)


def kernel(distance_matrix, direction_matrix, track_connectivity,
           dist_table, dir_table, track_bias):
    ct = (dist_table[:, None, :] + dir_table[None, :, :]).reshape(91, _H)
    ct = jnp.pad(ct, ((0, _TROW - 91), (0, 0)))
    ctab = jnp.concatenate([ct.T.reshape(-1), track_bias])

    out = _bias_kernel(
        ctab,
        distance_matrix.reshape(-1).astype(jnp.int32),
        direction_matrix.reshape(-1).astype(jnp.int32),
        track_connectivity.reshape(-1),
    )
    return out.reshape(_H, _N, _N)


# speedup vs baseline: 61.9478x; 2.8402x over previous
import functools

import jax
import jax.numpy as jnp
from jax import lax
from jax.experimental import pallas as pl
from jax.experimental.pallas import tpu as pltpu
from jax.experimental.pallas import tpu_sc as plsc

_H = 16
_N = 1024
_NW = 32
_RW = _N // _NW
_RPC = 2
_NCH = _RW // _RPC
_TROW = 96
_TSZ = _H * _TROW + _H


@functools.partial(
    pl.kernel,
    mesh=plsc.VectorSubcoreMesh(core_axis_name="c", subcore_axis_name="s"),
    out_type=jax.ShapeDtypeStruct((_H, _N, _N), jnp.float32),
    scratch_types=[
        pltpu.VMEM((_TSZ,), jnp.float32),
        pltpu.VMEM((2, _RPC, _N), jnp.int32),
        pltpu.VMEM((2, _RPC, _N), jnp.int32),
        pltpu.VMEM((2, _RPC, _N), jnp.float32),
        pltpu.VMEM((2, _H, _RPC, _N), jnp.float32),
        pltpu.SemaphoreType.DMA,
        pltpu.SemaphoreType.DMA,
        pltpu.SemaphoreType.DMA,
        pltpu.SemaphoreType.DMA,
    ],
    compiler_params=pltpu.CompilerParams(needs_layout_passes=False),
)
def _bias_kernel(ctab_hbm, dist_hbm, dir_hbm, track_hbm, out_hbm,
                 ctab_v, dist_v, dir_v, track_v, out_v,
                 in_sem0, in_sem1, out_sem0, out_sem1):
    in_sems = (in_sem0, in_sem1)
    out_sems = (out_sem0, out_sem1)
    wid = lax.axis_index("s") * 2 + lax.axis_index("c")
    pltpu.sync_copy(ctab_hbm, ctab_v)
    tb_splat = [
        plsc.load_gather(ctab_v, [jnp.full((16,), _H * _TROW + h, jnp.int32)])
        for h in range(_H)
    ]
    row0 = wid * _RW

    def in_copies(ci, buf):
        rb = row0 + ci * _RPC
        return [
            pltpu.make_async_copy(dist_hbm.at[pl.ds(rb, _RPC)],
                                  dist_v.at[buf], in_sems[buf]),
            pltpu.make_async_copy(dir_hbm.at[pl.ds(rb, _RPC)],
                                  dir_v.at[buf], in_sems[buf]),
            pltpu.make_async_copy(track_hbm.at[pl.ds(rb, _RPC)],
                                  track_v.at[buf], in_sems[buf]),
        ]

    def out_copies(ci, buf):
        rb = row0 + ci * _RPC
        return [
            pltpu.make_async_copy(out_v.at[buf, h],
                                  out_hbm.at[h, pl.ds(rb, _RPC)],
                                  out_sems[buf])
            for h in range(_H)
        ]

    for c in in_copies(0, 0):
        c.start()

    def chunk_pair(k, carry):
        for b in range(2):
            ci = 2 * k + b
            for c in in_copies(ci, b):
                c.wait()

            @pl.when(ci + 1 < _NCH)
            def _():
                for c in in_copies(ci + 1, 1 - b):
                    c.start()

            @pl.when(ci >= 2)
            def _():
                for c in out_copies(ci - 2, b):
                    c.wait()

            for r in range(_RPC):
                @plsc.parallel_loop(0, _N, step=16, unroll=2)
                def _(col):
                    d = dist_v[b, r, pl.ds(col, 16)]
                    rr = dir_v[b, r, pl.ds(col, 16)]
                    t = track_v[b, r, pl.ds(col, 16)]
                    c = jnp.minimum(d, 12) * 7 + rr
                    for h in range(_H):
                        g = plsc.load_gather(ctab_v, [c + h * _TROW])
                        out_v[b, h, r, pl.ds(col, 16)] = g + t * tb_splat[h]

            for c in out_copies(ci, b):
                c.start()
        return carry

    lax.fori_loop(0, _NCH // 2, chunk_pair, 0)
    for c in out_copies(_NCH - 2, 0):
        c.wait()
    for c in out_copies(_NCH - 1, 1):
        c.wait()


def kernel(distance_matrix, direction_matrix, track_connectivity,
           dist_table, dir_table, track_bias):
    ct = (dist_table[:, None, :] + dir_table[None, :, :]).reshape(91, _H)
    ct = jnp.pad(ct, ((0, _TROW - 91), (0, 0)))
    ctab = jnp.concatenate([ct.T.reshape(-1), track_bias])

    return _bias_kernel(
        ctab,
        distance_matrix.astype(jnp.int32),
        direction_matrix.astype(jnp.int32),
        track_connectivity,
    )


# speedup vs baseline: 71.1997x; 1.1494x over previous
import functools

import jax
import jax.numpy as jnp
from jax import lax
from jax.experimental import pallas as pl
from jax.experimental.pallas import tpu as pltpu
from jax.experimental.pallas import tpu_sc as plsc

_H = 16
_N = 1024
_NW = 32
_RW = _N // _NW
_RPC = 2
_NCH = _RW // _RPC
_TROW = 96
_TSZ = _H * _TROW + _H


@functools.partial(
    pl.kernel,
    mesh=plsc.VectorSubcoreMesh(core_axis_name="c", subcore_axis_name="s"),
    out_type=jax.ShapeDtypeStruct((_H, _N, _N), jnp.float32),
    scratch_types=[
        pltpu.VMEM((_TSZ,), jnp.float32),
        pltpu.VMEM((2, _RPC, _N), jnp.int32),
        pltpu.VMEM((2, _RPC, _N), jnp.int32),
        pltpu.VMEM((2, _RPC, _N), jnp.float32),
        pltpu.VMEM((2, _H, _RPC, _N), jnp.float32),
        pltpu.SemaphoreType.DMA,
        pltpu.SemaphoreType.DMA,
        pltpu.SemaphoreType.DMA,
        pltpu.SemaphoreType.DMA,
    ],
    compiler_params=pltpu.CompilerParams(needs_layout_passes=False),
)
def _bias_kernel(ctab_hbm, dist_hbm, dir_hbm, track_hbm, out_hbm,
                 ctab_v, dist_v, dir_v, track_v, out_v,
                 in_sem0, in_sem1, out_sem0, out_sem1):
    in_sems = (in_sem0, in_sem1)
    out_sems = (out_sem0, out_sem1)
    wid = lax.axis_index("s") * 2 + lax.axis_index("c")
    pltpu.sync_copy(ctab_hbm, ctab_v)
    tb_splat = [
        plsc.load_gather(ctab_v, [jnp.full((16,), _H * _TROW + h, jnp.int32)])
        for h in range(_H)
    ]
    row0 = wid * _RW

    def in_copies(ci, buf):
        rb = row0 + ci * _RPC
        return [
            pltpu.make_async_copy(dist_hbm.at[pl.ds(rb, _RPC)],
                                  dist_v.at[buf], in_sems[buf]),
            pltpu.make_async_copy(dir_hbm.at[pl.ds(rb, _RPC)],
                                  dir_v.at[buf], in_sems[buf]),
            pltpu.make_async_copy(track_hbm.at[pl.ds(rb, _RPC)],
                                  track_v.at[buf], in_sems[buf]),
        ]

    def out_copies(ci, buf):
        rb = row0 + ci * _RPC
        return [
            pltpu.make_async_copy(out_v.at[buf, h],
                                  out_hbm.at[h, pl.ds(rb, _RPC)],
                                  out_sems[buf])
            for h in range(_H)
        ]

    for c in in_copies(0, 0):
        c.start()

    def chunk_pair(k, carry):
        for b in range(2):
            ci = 2 * k + b
            for c in in_copies(ci, b):
                c.wait()

            @pl.when(ci + 1 < _NCH)
            def _():
                for c in in_copies(ci + 1, 1 - b):
                    c.start()

            @pl.when(ci >= 2)
            def _():
                for c in out_copies(ci - 2, b):
                    c.wait()

            for r in range(_RPC):
                @plsc.parallel_loop(0, _N, step=16, unroll=4)
                def _(col):
                    d = dist_v[b, r, pl.ds(col, 16)]
                    rr = dir_v[b, r, pl.ds(col, 16)]
                    t = track_v[b, r, pl.ds(col, 16)]
                    c = jnp.minimum(d, 12) * 7 + rr
                    for h in range(_H):
                        g = plsc.load_gather(ctab_v, [c + h * _TROW])
                        out_v[b, h, r, pl.ds(col, 16)] = g + t * tb_splat[h]

            for c in out_copies(ci, b):
                c.start()
        return carry

    lax.fori_loop(0, _NCH // 2, chunk_pair, 0)
    for c in out_copies(_NCH - 2, 0):
        c.wait()
    for c in out_copies(_NCH - 1, 1):
        c.wait()


def kernel(distance_matrix, direction_matrix, track_connectivity,
           dist_table, dir_table, track_bias):
    ct = (dist_table[:, None, :] + dir_table[None, :, :]).reshape(91, _H)
    ct = jnp.pad(ct, ((0, _TROW - 91), (0, 0)))
    ctab = jnp.concatenate([ct.T.reshape(-1), track_bias])

    return _bias_kernel(
        ctab,
        distance_matrix.astype(jnp.int32),
        direction_matrix.astype(jnp.int32),
        track_connectivity,
    )
